# baseline (speedup 1.0000x reference)
import functools

import jax
import jax.numpy as jnp
from jax.experimental import pallas as pl
from jax.experimental.pallas import tpu as pltpu

_MS_HIDDEN = 16


def _attn_kernel(row_ref, col_ref, cost_ref,
                 wq_ref, bq_ref, wk_ref, bk_ref, wv_ref, bv_ref,
                 wo_ref, bo_ref,
                 m1w0_ref, m1w1_ref, m1b_ref, m2w_ref, m2b_ref,
                 out_ref, *, head_dim):
    h = pl.program_id(1)
    row = row_ref[0]
    col = col_ref[0]
    w = cost_ref[0]

    dot = functools.partial(jnp.dot, preferred_element_type=jnp.float32,
                            precision=jax.lax.Precision.HIGHEST)
    qh = dot(row, wq_ref[0]) + bq_ref[0]
    kh = dot(col, wk_ref[0]) + bk_ref[0]
    vh = dot(col, wv_ref[0]) + bv_ref[0]

    logits = jax.lax.dot_general(
        qh, kh, (((1,), (1,)), ((), ())),
        preferred_element_type=jnp.float32,
        precision=jax.lax.Precision.HIGHEST) * (1.0 / (head_dim ** 0.5))

    def mlp_body(j, acc):
        a = m1w0_ref[h, j]
        b = m1w1_ref[h, j]
        c = m1b_ref[h, j]
        d = m2w_ref[h, j]
        hid = jnp.maximum(logits * a + w * b + c, 0.0)
        return acc + hid * d

    mlp = jax.lax.fori_loop(0, _MS_HIDDEN, mlp_body,
                            jnp.zeros_like(logits)) + m2b_ref[h, 0]

    mask = w > 0.0
    neg_inf = jnp.float32(-jnp.inf)
    masked = jnp.where(mask, mlp, neg_inf)
    gmax = jnp.max(masked, axis=1, keepdims=True)
    gmax = jnp.where(gmax == neg_inf, 0.0, gmax)
    ex = jnp.where(mask, jnp.exp(mlp - gmax), 0.0)
    denom = jnp.sum(ex, axis=1, keepdims=True)
    attn = ex / (denom + 1e-16)

    head_out = dot(attn, vh)
    contrib = dot(head_out, wo_ref[0])

    @pl.when(h == 0)
    def _init():
        out_ref[0] = contrib + bo_ref[0]

    @pl.when(h != 0)
    def _accum():
        out_ref[0] = out_ref[0] + contrib


def kernel(row_emb, col_emb, cost_mat, Wq, bq, Wk, bk, Wv, bv, Wo, bo,
           mix1_w, mix1_b, mix2_w, mix2_b):
    bs, row_cnt, emb = row_emb.shape
    col_cnt = col_emb.shape[1]
    num_heads = mix1_w.shape[0]
    head_dim = emb // num_heads

    wq_h = jnp.transpose(Wq.reshape(emb, num_heads, head_dim), (1, 0, 2))
    wk_h = jnp.transpose(Wk.reshape(emb, num_heads, head_dim), (1, 0, 2))
    wv_h = jnp.transpose(Wv.reshape(emb, num_heads, head_dim), (1, 0, 2))
    bq_h = bq.reshape(num_heads, 1, head_dim)
    bk_h = bk.reshape(num_heads, 1, head_dim)
    bv_h = bv.reshape(num_heads, 1, head_dim)
    wo_h = Wo.reshape(num_heads, head_dim, emb)
    bo_2d = bo.reshape(1, emb)
    m1w0 = mix1_w[:, 0, :]
    m1w1 = mix1_w[:, 1, :]
    m2w = mix2_w[:, :, 0]
    m2b = mix2_b

    def b_only(b, h):
        return (b, 0, 0)

    def h_only(b, h):
        return (h, 0, 0)

    smem = functools.partial(pl.BlockSpec, memory_space=pltpu.SMEM)

    out = pl.pallas_call(
        functools.partial(_attn_kernel, head_dim=head_dim),
        grid=(bs, num_heads),
        in_specs=[
            pl.BlockSpec((1, row_cnt, emb), b_only),
            pl.BlockSpec((1, col_cnt, emb), b_only),
            pl.BlockSpec((1, row_cnt, col_cnt), b_only),
            pl.BlockSpec((1, emb, head_dim), h_only),
            pl.BlockSpec((1, 1, head_dim), h_only),
            pl.BlockSpec((1, emb, head_dim), h_only),
            pl.BlockSpec((1, 1, head_dim), h_only),
            pl.BlockSpec((1, emb, head_dim), h_only),
            pl.BlockSpec((1, 1, head_dim), h_only),
            pl.BlockSpec((1, head_dim, emb), h_only),
            pl.BlockSpec((1, emb), lambda b, h: (0, 0)),
            smem((num_heads, _MS_HIDDEN), lambda b, h: (0, 0)),
            smem((num_heads, _MS_HIDDEN), lambda b, h: (0, 0)),
            smem((num_heads, _MS_HIDDEN), lambda b, h: (0, 0)),
            smem((num_heads, _MS_HIDDEN), lambda b, h: (0, 0)),
            smem((num_heads, 1), lambda b, h: (0, 0)),
        ],
        out_specs=pl.BlockSpec((1, row_cnt, emb), b_only),
        out_shape=jax.ShapeDtypeStruct((bs, row_cnt, emb), jnp.float32),
        compiler_params=pltpu.CompilerParams(
            dimension_semantics=("parallel", "arbitrary")),
    )(row_emb, col_emb, cost_mat,
      wq_h, bq_h, wk_h, bk_h, wv_h, bv_h, wo_h, bo_2d,
      m1w0, m1w1, mix1_b, m2w, m2b)
    return out


# speedup vs baseline: 1.8244x; 1.8244x over previous
import functools

import jax
import jax.numpy as jnp
from jax.experimental import pallas as pl
from jax.experimental.pallas import tpu as pltpu

_MS_HIDDEN = 16


def _attn_kernel(row_ref, col_ref, cost_ref,
                 wq_ref, bq_ref, wk_ref, bk_ref, wv_ref, bv_ref,
                 wo_ref, bo_ref,
                 m1w0_ref, m1w1_ref, m1b_ref, m2w_ref, m2b_ref,
                 out_ref, *, head_dim):
    h = pl.program_id(1)
    row = row_ref[0]
    col = col_ref[0]
    w = cost_ref[0]

    dot = functools.partial(jnp.dot, preferred_element_type=jnp.float32,
                            precision=jax.lax.Precision.HIGHEST)
    qh = dot(row, wq_ref[0]) + bq_ref[0]
    kh = dot(col, wk_ref[0]) + bk_ref[0]
    vh = dot(col, wv_ref[0]) + bv_ref[0]

    logits = jax.lax.dot_general(
        qh, kh, (((1,), (1,)), ((), ())),
        preferred_element_type=jnp.float32,
        precision=jax.lax.Precision.HIGHEST) * (1.0 / (head_dim ** 0.5))

    mlp = jnp.full_like(logits, 0.0)
    for j in range(_MS_HIDDEN):
        a = m1w0_ref[h, j]
        b = m1w1_ref[h, j]
        c = m1b_ref[h, j]
        d = m2w_ref[h, j]
        hid = jnp.maximum(logits * a + w * b + c, 0.0)
        mlp = mlp + hid * d
    mlp = mlp + m2b_ref[h, 0]

    mask = w > 0.0
    neg_inf = jnp.float32(-jnp.inf)
    masked = jnp.where(mask, mlp, neg_inf)
    gmax = jnp.max(masked, axis=1, keepdims=True)
    gmax = jnp.where(gmax == neg_inf, 0.0, gmax)
    ex = jnp.where(mask, jnp.exp(mlp - gmax), 0.0)
    denom = jnp.sum(ex, axis=1, keepdims=True)
    attn = ex / (denom + 1e-16)

    head_out = dot(attn, vh)
    contrib = dot(head_out, wo_ref[0])

    @pl.when(h == 0)
    def _init():
        out_ref[0] = contrib + bo_ref[0]

    @pl.when(h != 0)
    def _accum():
        out_ref[0] = out_ref[0] + contrib


def kernel(row_emb, col_emb, cost_mat, Wq, bq, Wk, bk, Wv, bv, Wo, bo,
           mix1_w, mix1_b, mix2_w, mix2_b):
    bs, row_cnt, emb = row_emb.shape
    col_cnt = col_emb.shape[1]
    num_heads = mix1_w.shape[0]
    head_dim = emb // num_heads

    wq_h = jnp.transpose(Wq.reshape(emb, num_heads, head_dim), (1, 0, 2))
    wk_h = jnp.transpose(Wk.reshape(emb, num_heads, head_dim), (1, 0, 2))
    wv_h = jnp.transpose(Wv.reshape(emb, num_heads, head_dim), (1, 0, 2))
    bq_h = bq.reshape(num_heads, 1, head_dim)
    bk_h = bk.reshape(num_heads, 1, head_dim)
    bv_h = bv.reshape(num_heads, 1, head_dim)
    wo_h = Wo.reshape(num_heads, head_dim, emb)
    bo_2d = bo.reshape(1, emb)
    m1w0 = mix1_w[:, 0, :]
    m1w1 = mix1_w[:, 1, :]
    m2w = mix2_w[:, :, 0]
    m2b = mix2_b

    def b_only(b, h):
        return (b, 0, 0)

    def h_only(b, h):
        return (h, 0, 0)

    smem = functools.partial(pl.BlockSpec, memory_space=pltpu.SMEM)

    out = pl.pallas_call(
        functools.partial(_attn_kernel, head_dim=head_dim),
        grid=(bs, num_heads),
        in_specs=[
            pl.BlockSpec((1, row_cnt, emb), b_only),
            pl.BlockSpec((1, col_cnt, emb), b_only),
            pl.BlockSpec((1, row_cnt, col_cnt), b_only),
            pl.BlockSpec((1, emb, head_dim), h_only),
            pl.BlockSpec((1, 1, head_dim), h_only),
            pl.BlockSpec((1, emb, head_dim), h_only),
            pl.BlockSpec((1, 1, head_dim), h_only),
            pl.BlockSpec((1, emb, head_dim), h_only),
            pl.BlockSpec((1, 1, head_dim), h_only),
            pl.BlockSpec((1, head_dim, emb), h_only),
            pl.BlockSpec((1, emb), lambda b, h: (0, 0)),
            smem((num_heads, _MS_HIDDEN), lambda b, h: (0, 0)),
            smem((num_heads, _MS_HIDDEN), lambda b, h: (0, 0)),
            smem((num_heads, _MS_HIDDEN), lambda b, h: (0, 0)),
            smem((num_heads, _MS_HIDDEN), lambda b, h: (0, 0)),
            smem((num_heads, 1), lambda b, h: (0, 0)),
        ],
        out_specs=pl.BlockSpec((1, row_cnt, emb), b_only),
        out_shape=jax.ShapeDtypeStruct((bs, row_cnt, emb), jnp.float32),
        compiler_params=pltpu.CompilerParams(
            dimension_semantics=("parallel", "arbitrary")),
    )(row_emb, col_emb, cost_mat,
      wq_h, bq_h, wk_h, bk_h, wv_h, bv_h, wo_h, bo_2d,
      m1w0, m1w1, mix1_b, m2w, m2b)
    return out


# speedup vs baseline: 2.2487x; 1.2326x over previous
import functools

import jax
import jax.numpy as jnp
from jax.experimental import pallas as pl
from jax.experimental.pallas import tpu as pltpu

_MS_HIDDEN = 16


def _attn_kernel(row_ref, col_ref, cost_ref,
                 wq_ref, bq_ref, wk_ref, bk_ref, wv_ref, bv_ref,
                 wo_ref, bo_ref,
                 m1w0_ref, m1w1_ref, m1b_ref, m2w_ref, m2b_ref,
                 out_ref, *, head_dim):
    h = pl.program_id(1)
    row = row_ref[0]
    col = col_ref[0]
    w = cost_ref[0]

    dot_hi = functools.partial(jnp.dot, preferred_element_type=jnp.float32,
                               precision=jax.lax.Precision.HIGHEST)
    dot = functools.partial(jnp.dot, preferred_element_type=jnp.float32)
    qh = dot_hi(row, wq_ref[0]) + bq_ref[0]
    kh = dot_hi(col, wk_ref[0]) + bk_ref[0]
    vh = dot(col, wv_ref[0]) + bv_ref[0]

    logits = jax.lax.dot_general(
        qh, kh, (((1,), (1,)), ((), ())),
        preferred_element_type=jnp.float32,
        precision=jax.lax.Precision.HIGHEST) * (1.0 / (head_dim ** 0.5))

    mlp = jnp.full_like(logits, 0.0)
    for j in range(_MS_HIDDEN):
        a = m1w0_ref[h, j]
        b = m1w1_ref[h, j]
        c = m1b_ref[h, j]
        d = m2w_ref[h, j]
        hid = jnp.maximum(logits * a + w * b + c, 0.0)
        mlp = mlp + hid * d
    mlp = mlp + m2b_ref[h, 0]

    mask = w > 0.0
    neg_inf = jnp.float32(-jnp.inf)
    masked = jnp.where(mask, mlp, neg_inf)
    gmax = jnp.max(masked, axis=1, keepdims=True)
    gmax = jnp.where(gmax == neg_inf, 0.0, gmax)
    ex = jnp.where(mask, jnp.exp(mlp - gmax), 0.0)
    denom = jnp.sum(ex, axis=1, keepdims=True)
    attn = ex / (denom + 1e-16)

    head_out = dot(attn, vh)
    contrib = dot(head_out, wo_ref[0])

    @pl.when(h == 0)
    def _init():
        out_ref[0] = contrib + bo_ref[0]

    @pl.when(h != 0)
    def _accum():
        out_ref[0] = out_ref[0] + contrib


def kernel(row_emb, col_emb, cost_mat, Wq, bq, Wk, bk, Wv, bv, Wo, bo,
           mix1_w, mix1_b, mix2_w, mix2_b):
    bs, row_cnt, emb = row_emb.shape
    col_cnt = col_emb.shape[1]
    num_heads = mix1_w.shape[0]
    head_dim = emb // num_heads

    wq_h = jnp.transpose(Wq.reshape(emb, num_heads, head_dim), (1, 0, 2))
    wk_h = jnp.transpose(Wk.reshape(emb, num_heads, head_dim), (1, 0, 2))
    wv_h = jnp.transpose(Wv.reshape(emb, num_heads, head_dim), (1, 0, 2))
    bq_h = bq.reshape(num_heads, 1, head_dim)
    bk_h = bk.reshape(num_heads, 1, head_dim)
    bv_h = bv.reshape(num_heads, 1, head_dim)
    wo_h = Wo.reshape(num_heads, head_dim, emb)
    bo_2d = bo.reshape(1, emb)
    m1w0 = mix1_w[:, 0, :]
    m1w1 = mix1_w[:, 1, :]
    m2w = mix2_w[:, :, 0]
    m2b = mix2_b

    def b_only(b, h):
        return (b, 0, 0)

    def h_only(b, h):
        return (h, 0, 0)

    smem = functools.partial(pl.BlockSpec, memory_space=pltpu.SMEM)

    out = pl.pallas_call(
        functools.partial(_attn_kernel, head_dim=head_dim),
        grid=(bs, num_heads),
        in_specs=[
            pl.BlockSpec((1, row_cnt, emb), b_only),
            pl.BlockSpec((1, col_cnt, emb), b_only),
            pl.BlockSpec((1, row_cnt, col_cnt), b_only),
            pl.BlockSpec((1, emb, head_dim), h_only),
            pl.BlockSpec((1, 1, head_dim), h_only),
            pl.BlockSpec((1, emb, head_dim), h_only),
            pl.BlockSpec((1, 1, head_dim), h_only),
            pl.BlockSpec((1, emb, head_dim), h_only),
            pl.BlockSpec((1, 1, head_dim), h_only),
            pl.BlockSpec((1, head_dim, emb), h_only),
            pl.BlockSpec((1, emb), lambda b, h: (0, 0)),
            smem((num_heads, _MS_HIDDEN), lambda b, h: (0, 0)),
            smem((num_heads, _MS_HIDDEN), lambda b, h: (0, 0)),
            smem((num_heads, _MS_HIDDEN), lambda b, h: (0, 0)),
            smem((num_heads, _MS_HIDDEN), lambda b, h: (0, 0)),
            smem((num_heads, 1), lambda b, h: (0, 0)),
        ],
        out_specs=pl.BlockSpec((1, row_cnt, emb), b_only),
        out_shape=jax.ShapeDtypeStruct((bs, row_cnt, emb), jnp.float32),
        compiler_params=pltpu.CompilerParams(
            dimension_semantics=("parallel", "arbitrary")),
    )(row_emb, col_emb, cost_mat,
      wq_h, bq_h, wk_h, bk_h, wv_h, bv_h, wo_h, bo_2d,
      m1w0, m1w1, mix1_b, m2w, m2b)
    return out


# speedup vs baseline: 2.6910x; 1.1967x over previous
import functools

import jax
import jax.numpy as jnp
from jax.experimental import pallas as pl
from jax.experimental.pallas import tpu as pltpu

_MS_HIDDEN = 16


def _attn_kernel(row_ref, col_ref, cost_ref,
                 wq_ref, bq_ref, wk_ref, bk_ref, wv_ref, bv_ref,
                 wo_ref, bo_ref,
                 m1w0_ref, m1w1_ref, m1b_ref, m2w_ref, m2b_ref,
                 out_ref, *, head_dim):
    h = pl.program_id(1)
    row = row_ref[0]
    col = col_ref[0]
    w = cost_ref[0]

    dot = functools.partial(jnp.dot, preferred_element_type=jnp.float32)
    qh = dot(row, wq_ref[0]) + bq_ref[0]
    kh = dot(col, wk_ref[0]) + bk_ref[0]
    vh = dot(col, wv_ref[0]) + bv_ref[0]

    logits = jax.lax.dot_general(
        qh, kh, (((1,), (1,)), ((), ())),
        preferred_element_type=jnp.float32) * (1.0 / (head_dim ** 0.5))

    mlp = jnp.full_like(logits, 0.0)
    for j in range(_MS_HIDDEN):
        a = m1w0_ref[h, j]
        b = m1w1_ref[h, j]
        c = m1b_ref[h, j]
        d = m2w_ref[h, j]
        hid = jnp.maximum(logits * a + w * b + c, 0.0)
        mlp = mlp + hid * d
    mlp = mlp + m2b_ref[h, 0]

    neg_inf = jnp.float32(-jnp.inf)
    masked = jnp.where(w > 0.0, mlp, neg_inf)
    gmax = jnp.max(masked, axis=1, keepdims=True)
    gmax = jnp.where(gmax == neg_inf, 0.0, gmax)
    ex = jnp.exp(masked - gmax)
    denom = jnp.sum(ex, axis=1, keepdims=True)
    attn = ex / (denom + 1e-16)

    head_out = dot(attn, vh)
    contrib = dot(head_out, wo_ref[0])

    @pl.when(h == 0)
    def _init():
        out_ref[0] = contrib + bo_ref[0]

    @pl.when(h != 0)
    def _accum():
        out_ref[0] = out_ref[0] + contrib


def kernel(row_emb, col_emb, cost_mat, Wq, bq, Wk, bk, Wv, bv, Wo, bo,
           mix1_w, mix1_b, mix2_w, mix2_b):
    bs, row_cnt, emb = row_emb.shape
    col_cnt = col_emb.shape[1]
    num_heads = mix1_w.shape[0]
    head_dim = emb // num_heads

    wq_h = jnp.transpose(Wq.reshape(emb, num_heads, head_dim), (1, 0, 2))
    wk_h = jnp.transpose(Wk.reshape(emb, num_heads, head_dim), (1, 0, 2))
    wv_h = jnp.transpose(Wv.reshape(emb, num_heads, head_dim), (1, 0, 2))
    bq_h = bq.reshape(num_heads, 1, head_dim)
    bk_h = bk.reshape(num_heads, 1, head_dim)
    bv_h = bv.reshape(num_heads, 1, head_dim)
    wo_h = Wo.reshape(num_heads, head_dim, emb)
    bo_2d = bo.reshape(1, emb)
    m1w0 = mix1_w[:, 0, :]
    m1w1 = mix1_w[:, 1, :]
    m2w = mix2_w[:, :, 0]
    m2b = mix2_b

    def b_only(b, h):
        return (b, 0, 0)

    def h_only(b, h):
        return (h, 0, 0)

    smem = functools.partial(pl.BlockSpec, memory_space=pltpu.SMEM)

    out = pl.pallas_call(
        functools.partial(_attn_kernel, head_dim=head_dim),
        grid=(bs, num_heads),
        in_specs=[
            pl.BlockSpec((1, row_cnt, emb), b_only),
            pl.BlockSpec((1, col_cnt, emb), b_only),
            pl.BlockSpec((1, row_cnt, col_cnt), b_only),
            pl.BlockSpec((1, emb, head_dim), h_only),
            pl.BlockSpec((1, 1, head_dim), h_only),
            pl.BlockSpec((1, emb, head_dim), h_only),
            pl.BlockSpec((1, 1, head_dim), h_only),
            pl.BlockSpec((1, emb, head_dim), h_only),
            pl.BlockSpec((1, 1, head_dim), h_only),
            pl.BlockSpec((1, head_dim, emb), h_only),
            pl.BlockSpec((1, emb), lambda b, h: (0, 0)),
            smem((num_heads, _MS_HIDDEN), lambda b, h: (0, 0)),
            smem((num_heads, _MS_HIDDEN), lambda b, h: (0, 0)),
            smem((num_heads, _MS_HIDDEN), lambda b, h: (0, 0)),
            smem((num_heads, _MS_HIDDEN), lambda b, h: (0, 0)),
            smem((num_heads, 1), lambda b, h: (0, 0)),
        ],
        out_specs=pl.BlockSpec((1, row_cnt, emb), b_only),
        out_shape=jax.ShapeDtypeStruct((bs, row_cnt, emb), jnp.float32),
        compiler_params=pltpu.CompilerParams(
            dimension_semantics=("parallel", "arbitrary")),
    )(row_emb, col_emb, cost_mat,
      wq_h, bq_h, wk_h, bk_h, wv_h, bv_h, wo_h, bo_2d,
      m1w0, m1w1, mix1_b, m2w, m2b)
    return out


# speedup vs baseline: 2.7394x; 1.0180x over previous
import functools

import jax
import jax.numpy as jnp
from jax.experimental import pallas as pl
from jax.experimental.pallas import tpu as pltpu

_MS_HIDDEN = 16


def _attn_kernel(row_ref, col_ref, cost_ref,
                 wq_ref, bq_ref, wk_ref, bk_ref, wv_ref, bv_ref,
                 wo_ref, bo_ref,
                 m1w0_ref, m1w1_ref, m1b_ref, m2w_ref, m2b_ref,
                 out_ref, *, head_dim):
    h = pl.program_id(1)
    row = row_ref[0]
    col = col_ref[0]
    w = cost_ref[0]

    dot = functools.partial(jnp.dot, preferred_element_type=jnp.float32)
    qh = (dot(row, wq_ref[0]) + bq_ref[0]) * (head_dim ** -0.5)
    kh = dot(col, wk_ref[0]) + bk_ref[0]
    vh = dot(col, wv_ref[0]) + bv_ref[0]

    logits = jax.lax.dot_general(
        qh, kh, (((1,), (1,)), ((), ())),
        preferred_element_type=jnp.float32)

    mlp = jnp.full_like(logits, 0.0)
    for j in range(_MS_HIDDEN):
        a = m1w0_ref[h, j]
        b = m1w1_ref[h, j]
        c = m1b_ref[h, j]
        d = m2w_ref[h, j]
        hid = jnp.maximum(logits * a + w * b + c, 0.0)
        mlp = mlp + hid * d
    mlp = mlp + m2b_ref[h, 0]

    neg_inf = jnp.float32(-jnp.inf)
    masked = jnp.where(w > 0.0, mlp, neg_inf)
    gmax = jnp.max(masked, axis=1, keepdims=True)
    gmax = jnp.where(gmax == neg_inf, 0.0, gmax)
    ex = jnp.exp(masked - gmax)
    denom = jnp.sum(ex, axis=1, keepdims=True)
    head_out = dot(ex, vh) / (denom + 1e-16)
    contrib = dot(head_out, wo_ref[0])

    @pl.when(h == 0)
    def _init():
        out_ref[0] = contrib + bo_ref[0]

    @pl.when(h != 0)
    def _accum():
        out_ref[0] = out_ref[0] + contrib


def kernel(row_emb, col_emb, cost_mat, Wq, bq, Wk, bk, Wv, bv, Wo, bo,
           mix1_w, mix1_b, mix2_w, mix2_b):
    bs, row_cnt, emb = row_emb.shape
    col_cnt = col_emb.shape[1]
    num_heads = mix1_w.shape[0]
    head_dim = emb // num_heads

    wq_h = jnp.transpose(Wq.reshape(emb, num_heads, head_dim), (1, 0, 2))
    wk_h = jnp.transpose(Wk.reshape(emb, num_heads, head_dim), (1, 0, 2))
    wv_h = jnp.transpose(Wv.reshape(emb, num_heads, head_dim), (1, 0, 2))
    bq_h = bq.reshape(num_heads, 1, head_dim)
    bk_h = bk.reshape(num_heads, 1, head_dim)
    bv_h = bv.reshape(num_heads, 1, head_dim)
    wo_h = Wo.reshape(num_heads, head_dim, emb)
    bo_2d = bo.reshape(1, emb)
    m1w0 = mix1_w[:, 0, :]
    m1w1 = mix1_w[:, 1, :]
    m2w = mix2_w[:, :, 0]
    m2b = mix2_b

    def b_only(b, h):
        return (b, 0, 0)

    def h_only(b, h):
        return (h, 0, 0)

    smem = functools.partial(pl.BlockSpec, memory_space=pltpu.SMEM)

    out = pl.pallas_call(
        functools.partial(_attn_kernel, head_dim=head_dim),
        grid=(bs, num_heads),
        in_specs=[
            pl.BlockSpec((1, row_cnt, emb), b_only),
            pl.BlockSpec((1, col_cnt, emb), b_only),
            pl.BlockSpec((1, row_cnt, col_cnt), b_only),
            pl.BlockSpec((1, emb, head_dim), h_only),
            pl.BlockSpec((1, 1, head_dim), h_only),
            pl.BlockSpec((1, emb, head_dim), h_only),
            pl.BlockSpec((1, 1, head_dim), h_only),
            pl.BlockSpec((1, emb, head_dim), h_only),
            pl.BlockSpec((1, 1, head_dim), h_only),
            pl.BlockSpec((1, head_dim, emb), h_only),
            pl.BlockSpec((1, emb), lambda b, h: (0, 0)),
            smem((num_heads, _MS_HIDDEN), lambda b, h: (0, 0)),
            smem((num_heads, _MS_HIDDEN), lambda b, h: (0, 0)),
            smem((num_heads, _MS_HIDDEN), lambda b, h: (0, 0)),
            smem((num_heads, _MS_HIDDEN), lambda b, h: (0, 0)),
            smem((num_heads, 1), lambda b, h: (0, 0)),
        ],
        out_specs=pl.BlockSpec((1, row_cnt, emb), b_only),
        out_shape=jax.ShapeDtypeStruct((bs, row_cnt, emb), jnp.float32),
        compiler_params=pltpu.CompilerParams(
            dimension_semantics=("parallel", "arbitrary")),
    )(row_emb, col_emb, cost_mat,
      wq_h, bq_h, wk_h, bk_h, wv_h, bv_h, wo_h, bo_2d,
      m1w0, m1w1, mix1_b, m2w, m2b)
    return out


# speedup vs baseline: 2.8764x; 1.0500x over previous
import functools

import jax
import jax.numpy as jnp
from jax.experimental import pallas as pl
from jax.experimental.pallas import tpu as pltpu

_MS_HIDDEN = 16
_HEADS_PER_STEP = 2


def _attn_kernel(row_ref, col_ref, cost_ref,
                 wq_ref, bq_ref, wk_ref, bk_ref, wv_ref, bv_ref,
                 wo_ref, bo_ref,
                 m1w0_ref, m1w1_ref, m1b_ref, m2w_ref, m2b_ref,
                 out_ref, *, head_dim, heads_per_step):
    g = pl.program_id(1)
    row = row_ref[0]
    col = col_ref[0]
    w = cost_ref[0]

    dot = functools.partial(jnp.dot, preferred_element_type=jnp.float32)
    contrib_total = None
    for hh in range(heads_per_step):
        h = g * heads_per_step + hh
        qh = (dot(row, wq_ref[hh]) + bq_ref[hh]) * (head_dim ** -0.5)
        kh = dot(col, wk_ref[hh]) + bk_ref[hh]
        vh = dot(col, wv_ref[hh]) + bv_ref[hh]

        logits = jax.lax.dot_general(
            qh, kh, (((1,), (1,)), ((), ())),
            preferred_element_type=jnp.float32)

        mlp = jnp.full_like(logits, 0.0)
        for j in range(_MS_HIDDEN):
            a = m1w0_ref[h, j]
            b = m1w1_ref[h, j]
            c = m1b_ref[h, j]
            d = m2w_ref[h, j]
            hid = jnp.maximum(logits * a + w * b + c, 0.0)
            mlp = mlp + hid * d
        mlp = mlp + m2b_ref[h, 0]

        neg_inf = jnp.float32(-jnp.inf)
        masked = jnp.where(w > 0.0, mlp, neg_inf)
        gmax = jnp.max(masked, axis=1, keepdims=True)
        gmax = jnp.where(gmax == neg_inf, 0.0, gmax)
        ex = jnp.exp(masked - gmax)
        denom = jnp.sum(ex, axis=1, keepdims=True)
        head_out = dot(ex, vh) / (denom + 1e-16)
        contrib = dot(head_out, wo_ref[hh])
        contrib_total = contrib if contrib_total is None else contrib_total + contrib

    @pl.when(g == 0)
    def _init():
        out_ref[0] = contrib_total + bo_ref[0]

    @pl.when(g != 0)
    def _accum():
        out_ref[0] = out_ref[0] + contrib_total


def kernel(row_emb, col_emb, cost_mat, Wq, bq, Wk, bk, Wv, bv, Wo, bo,
           mix1_w, mix1_b, mix2_w, mix2_b):
    bs, row_cnt, emb = row_emb.shape
    col_cnt = col_emb.shape[1]
    num_heads = mix1_w.shape[0]
    head_dim = emb // num_heads
    hps = _HEADS_PER_STEP if num_heads % _HEADS_PER_STEP == 0 else 1

    wq_h = jnp.transpose(Wq.reshape(emb, num_heads, head_dim), (1, 0, 2))
    wk_h = jnp.transpose(Wk.reshape(emb, num_heads, head_dim), (1, 0, 2))
    wv_h = jnp.transpose(Wv.reshape(emb, num_heads, head_dim), (1, 0, 2))
    bq_h = bq.reshape(num_heads, 1, head_dim)
    bk_h = bk.reshape(num_heads, 1, head_dim)
    bv_h = bv.reshape(num_heads, 1, head_dim)
    wo_h = Wo.reshape(num_heads, head_dim, emb)
    bo_2d = bo.reshape(1, emb)
    m1w0 = mix1_w[:, 0, :]
    m1w1 = mix1_w[:, 1, :]
    m2w = mix2_w[:, :, 0]
    m2b = mix2_b

    def b_only(b, g):
        return (b, 0, 0)

    def g_only(b, g):
        return (g, 0, 0)

    smem = functools.partial(pl.BlockSpec, memory_space=pltpu.SMEM)

    out = pl.pallas_call(
        functools.partial(_attn_kernel, head_dim=head_dim,
                          heads_per_step=hps),
        grid=(bs, num_heads // hps),
        in_specs=[
            pl.BlockSpec((1, row_cnt, emb), b_only),
            pl.BlockSpec((1, col_cnt, emb), b_only),
            pl.BlockSpec((1, row_cnt, col_cnt), b_only),
            pl.BlockSpec((hps, emb, head_dim), g_only),
            pl.BlockSpec((hps, 1, head_dim), g_only),
            pl.BlockSpec((hps, emb, head_dim), g_only),
            pl.BlockSpec((hps, 1, head_dim), g_only),
            pl.BlockSpec((hps, emb, head_dim), g_only),
            pl.BlockSpec((hps, 1, head_dim), g_only),
            pl.BlockSpec((hps, head_dim, emb), g_only),
            pl.BlockSpec((1, emb), lambda b, g: (0, 0)),
            smem((num_heads, _MS_HIDDEN), lambda b, g: (0, 0)),
            smem((num_heads, _MS_HIDDEN), lambda b, g: (0, 0)),
            smem((num_heads, _MS_HIDDEN), lambda b, g: (0, 0)),
            smem((num_heads, _MS_HIDDEN), lambda b, g: (0, 0)),
            smem((num_heads, 1), lambda b, g: (0, 0)),
        ],
        out_specs=pl.BlockSpec((1, row_cnt, emb), b_only),
        out_shape=jax.ShapeDtypeStruct((bs, row_cnt, emb), jnp.float32),
        compiler_params=pltpu.CompilerParams(
            dimension_semantics=("parallel", "arbitrary")),
    )(row_emb, col_emb, cost_mat,
      wq_h, bq_h, wk_h, bk_h, wv_h, bv_h, wo_h, bo_2d,
      m1w0, m1w1, mix1_b, m2w, m2b)
    return out


# speedup vs baseline: 2.9113x; 1.0122x over previous
import functools

import jax
import jax.numpy as jnp
from jax.experimental import pallas as pl
from jax.experimental.pallas import tpu as pltpu

_MS_HIDDEN = 16
_HEADS_PER_STEP = 4


def _attn_kernel(row_ref, col_ref, cost_ref,
                 wq_ref, bq_ref, wk_ref, bk_ref, wv_ref, bv_ref,
                 wo_ref, bo_ref,
                 m1w0_ref, m1w1_ref, m1b_ref, m2w_ref, m2b_ref,
                 out_ref, *, head_dim, heads_per_step):
    g = pl.program_id(1)
    row = row_ref[0]
    col = col_ref[0]
    w = cost_ref[0]

    dot = functools.partial(jnp.dot, preferred_element_type=jnp.float32)
    contrib_total = None
    for hh in range(heads_per_step):
        h = g * heads_per_step + hh
        qh = (dot(row, wq_ref[hh]) + bq_ref[hh]) * (head_dim ** -0.5)
        kh = dot(col, wk_ref[hh]) + bk_ref[hh]
        vh = dot(col, wv_ref[hh]) + bv_ref[hh]

        logits = jax.lax.dot_general(
            qh, kh, (((1,), (1,)), ((), ())),
            preferred_element_type=jnp.float32)

        mlp = jnp.full_like(logits, 0.0)
        for j in range(_MS_HIDDEN):
            a = m1w0_ref[h, j]
            b = m1w1_ref[h, j]
            c = m1b_ref[h, j]
            d = m2w_ref[h, j]
            hid = jnp.maximum(logits * a + w * b + c, 0.0)
            mlp = mlp + hid * d
        mlp = mlp + m2b_ref[h, 0]

        neg_inf = jnp.float32(-jnp.inf)
        masked = jnp.where(w > 0.0, mlp, neg_inf)
        gmax = jnp.max(masked, axis=1, keepdims=True)
        gmax = jnp.where(gmax == neg_inf, 0.0, gmax)
        ex = jnp.exp(masked - gmax)
        denom = jnp.sum(ex, axis=1, keepdims=True)
        head_out = dot(ex, vh) / (denom + 1e-16)
        contrib = dot(head_out, wo_ref[hh])
        contrib_total = contrib if contrib_total is None else contrib_total + contrib

    @pl.when(g == 0)
    def _init():
        out_ref[0] = contrib_total + bo_ref[0]

    @pl.when(g != 0)
    def _accum():
        out_ref[0] = out_ref[0] + contrib_total


def kernel(row_emb, col_emb, cost_mat, Wq, bq, Wk, bk, Wv, bv, Wo, bo,
           mix1_w, mix1_b, mix2_w, mix2_b):
    bs, row_cnt, emb = row_emb.shape
    col_cnt = col_emb.shape[1]
    num_heads = mix1_w.shape[0]
    head_dim = emb // num_heads
    hps = _HEADS_PER_STEP if num_heads % _HEADS_PER_STEP == 0 else 1

    wq_h = jnp.transpose(Wq.reshape(emb, num_heads, head_dim), (1, 0, 2))
    wk_h = jnp.transpose(Wk.reshape(emb, num_heads, head_dim), (1, 0, 2))
    wv_h = jnp.transpose(Wv.reshape(emb, num_heads, head_dim), (1, 0, 2))
    bq_h = bq.reshape(num_heads, 1, head_dim)
    bk_h = bk.reshape(num_heads, 1, head_dim)
    bv_h = bv.reshape(num_heads, 1, head_dim)
    wo_h = Wo.reshape(num_heads, head_dim, emb)
    bo_2d = bo.reshape(1, emb)
    m1w0 = mix1_w[:, 0, :]
    m1w1 = mix1_w[:, 1, :]
    m2w = mix2_w[:, :, 0]
    m2b = mix2_b

    def b_only(b, g):
        return (b, 0, 0)

    def g_only(b, g):
        return (g, 0, 0)

    smem = functools.partial(pl.BlockSpec, memory_space=pltpu.SMEM)

    out = pl.pallas_call(
        functools.partial(_attn_kernel, head_dim=head_dim,
                          heads_per_step=hps),
        grid=(bs, num_heads // hps),
        in_specs=[
            pl.BlockSpec((1, row_cnt, emb), b_only),
            pl.BlockSpec((1, col_cnt, emb), b_only),
            pl.BlockSpec((1, row_cnt, col_cnt), b_only),
            pl.BlockSpec((hps, emb, head_dim), g_only),
            pl.BlockSpec((hps, 1, head_dim), g_only),
            pl.BlockSpec((hps, emb, head_dim), g_only),
            pl.BlockSpec((hps, 1, head_dim), g_only),
            pl.BlockSpec((hps, emb, head_dim), g_only),
            pl.BlockSpec((hps, 1, head_dim), g_only),
            pl.BlockSpec((hps, head_dim, emb), g_only),
            pl.BlockSpec((1, emb), lambda b, g: (0, 0)),
            smem((num_heads, _MS_HIDDEN), lambda b, g: (0, 0)),
            smem((num_heads, _MS_HIDDEN), lambda b, g: (0, 0)),
            smem((num_heads, _MS_HIDDEN), lambda b, g: (0, 0)),
            smem((num_heads, _MS_HIDDEN), lambda b, g: (0, 0)),
            smem((num_heads, 1), lambda b, g: (0, 0)),
        ],
        out_specs=pl.BlockSpec((1, row_cnt, emb), b_only),
        out_shape=jax.ShapeDtypeStruct((bs, row_cnt, emb), jnp.float32),
        compiler_params=pltpu.CompilerParams(
            dimension_semantics=("parallel", "arbitrary")),
    )(row_emb, col_emb, cost_mat,
      wq_h, bq_h, wk_h, bk_h, wv_h, bv_h, wo_h, bo_2d,
      m1w0, m1w1, mix1_b, m2w, m2b)
    return out
